# Initial kernel scaffold; baseline (speedup 1.0000x reference)
import jax
import jax.numpy as jnp
from jax.experimental import pallas as pl


def kernel(input_ids, table):
    raise NotImplementedError("write your pallas kernel here")


# speedup vs baseline: 3.9708x; 3.9708x over previous
import functools
import jax
import jax.numpy as jnp
from jax import lax
from jax.experimental import pallas as pl
from jax.experimental.pallas import tpu as pltpu
from jax.experimental.pallas import tpu_sc as plsc


def _build_sc_gather(N, V, D, T, CSZ, C):
    info = plsc.get_sparse_core_info()
    NC, NS, L = info.num_cores, info.num_subcores, info.num_lanes
    NW = NC * NS
    per_w = N // NW
    CH = 1024
    n_chunks = per_w // CH

    mesh = plsc.VectorSubcoreMesh(core_axis_name="c", subcore_axis_name="s")

    @functools.partial(
        pl.kernel,
        mesh=mesh,
        compiler_params=pltpu.CompilerParams(use_tc_tiling_on_sc=False),
        out_type=jax.ShapeDtypeStruct((N, D), jnp.float32),
        scratch_types=[
            pltpu.VMEM((CH,), jnp.int32),
            pltpu.VMEM((CH, D), jnp.float32),
            pltpu.SemaphoreType.DMA,
        ],
    )
    def k(idx_hbm, table_hbm, out_hbm, idx_v, rows_v, sem):
        wid = lax.axis_index("s") * NC + lax.axis_index("c")

        def chunk_body(ci, _):
            g = wid * n_chunks + ci
            start = g * CH
            c = (start // T) % C
            off = (c * CSZ).astype(jnp.int32)
            pltpu.sync_copy(idx_hbm.at[pl.ds(start, CH)], idx_v)

            def add_body(j, _):
                sl = pl.ds(j * L, L)
                idx_v[sl] = idx_v[sl] + off
                return 0

            lax.fori_loop(0, CH // L, add_body, 0, unroll=True)
            pltpu.async_copy(table_hbm.at[idx_v], rows_v, sem).wait()
            pltpu.sync_copy(rows_v, out_hbm.at[pl.ds(start, CH)])
            return 0

        lax.fori_loop(0, n_chunks, chunk_body, 0)

    return k


def kernel(input_ids, table):
    B_, C_, T_ = input_ids.shape
    V_, D_ = table.shape
    CSZ = V_ // C_
    N = B_ * C_ * T_
    flat_idx = input_ids.reshape(N).astype(jnp.int32)
    k = _build_sc_gather(N, V_, D_, T_, CSZ, C_)
    out = k(flat_idx, table)
    return out.reshape(B_, C_, T_, D_)


# speedup vs baseline: 4.0630x; 1.0232x over previous
import functools
import jax
import jax.numpy as jnp
from jax import lax
from jax.experimental import pallas as pl
from jax.experimental.pallas import tpu as pltpu
from jax.experimental.pallas import tpu_sc as plsc


def _build_sc_gather(N, V, D, T, CSZ, C):
    info = plsc.get_sparse_core_info()
    NC, NS, L = info.num_cores, info.num_subcores, info.num_lanes
    NW = NC * NS
    per_w = N // NW
    CH = 512
    n_chunks = per_w // CH

    mesh = plsc.VectorSubcoreMesh(core_axis_name="c", subcore_axis_name="s")

    @functools.partial(
        pl.kernel,
        mesh=mesh,
        compiler_params=pltpu.CompilerParams(use_tc_tiling_on_sc=False),
        out_type=jax.ShapeDtypeStruct((N, D), jnp.float32),
        scratch_types=[
            pltpu.VMEM((2, CH), jnp.int32),
            pltpu.VMEM((2, CH, D), jnp.float32),
            pltpu.SemaphoreType.DMA((2,)),
            pltpu.SemaphoreType.DMA((2,)),
        ],
    )
    def k(idx_hbm, table_hbm, out_hbm, idx_v, rows_v, gsem, osem):
        wid = lax.axis_index("s") * NC + lax.axis_index("c")
        base_chunk = wid * n_chunks

        def load_and_gather(ci, slot):
            g = base_chunk + ci
            start = g * CH
            c = (start // T) % C
            off = (c * CSZ).astype(jnp.int32)
            iv = idx_v.at[slot]
            pltpu.sync_copy(idx_hbm.at[pl.ds(start, CH)], iv)

            def add_body(j, _):
                sl = pl.ds(j * L, L)
                iv[sl] = iv[sl] + off
                return 0

            lax.fori_loop(0, CH // L, add_body, 0, unroll=True)
            pltpu.async_copy(
                table_hbm.at[idx_v.at[slot]], rows_v.at[slot], gsem.at[slot]
            )

        def write_out(ci, slot):
            start = (base_chunk + ci) * CH
            pltpu.async_copy(
                rows_v.at[slot], out_hbm.at[pl.ds(start, CH)], osem.at[slot]
            )

        load_and_gather(0, 0)
        for g in range(n_chunks):
            slot = g % 2
            nslot = (g + 1) % 2
            if g + 1 < n_chunks:
                if g >= 1:
                    pltpu.make_async_copy(
                        rows_v.at[nslot],
                        out_hbm.at[pl.ds(0, CH)],
                        osem.at[nslot],
                    ).wait()
                load_and_gather(g + 1, nslot)
            pltpu.make_async_copy(
                table_hbm.at[idx_v.at[slot]], rows_v.at[slot], gsem.at[slot]
            ).wait()
            write_out(g, slot)
        pltpu.make_async_copy(
            rows_v.at[0], out_hbm.at[pl.ds(0, CH)], osem.at[0]
        ).wait()
        pltpu.make_async_copy(
            rows_v.at[1], out_hbm.at[pl.ds(0, CH)], osem.at[1]
        ).wait()

    return k


def kernel(input_ids, table):
    B_, C_, T_ = input_ids.shape
    V_, D_ = table.shape
    CSZ = V_ // C_
    N = B_ * C_ * T_
    flat_idx = input_ids.reshape(N).astype(jnp.int32)
    k = _build_sc_gather(N, V_, D_, T_, CSZ, C_)
    out = k(flat_idx, table)
    return out.reshape(B_, C_, T_, D_)
